# baseline (speedup 1.0000x reference)
import functools

import jax
import jax.numpy as jnp
from jax import lax
from jax.experimental import pallas as pl
from jax.experimental.pallas import tpu as pltpu
from jax.experimental.pallas import tpu_sc as plsc

BATCH = 16384
HIST = 20
DIM = 32
TOTAL = BATCH * HIST

NC = 2
NS = 16
NW = NC * NS
BPW = TOTAL // NW

CHUNK = 128
GC = 4
NGROUP = BPW // (CHUNK * GC)
CPH = BATCH // CHUNK

_mesh = plsc.VectorSubcoreMesh(core_axis_name="c", subcore_axis_name="s")


@functools.partial(
    pl.kernel,
    mesh=_mesh,
    out_type=jax.ShapeDtypeStruct((HIST, DIM // 8, BATCH // 128, 8, 128), jnp.float32),
    scratch_types=(
        [
            pltpu.VMEM((2, GC, CHUNK), jnp.int32),
            pltpu.VMEM((2, GC, CHUNK, DIM), jnp.float32),
            pltpu.VMEM((2, GC, DIM, CHUNK), jnp.float32),
        ]
        + [pltpu.SemaphoreType.DMA] * 6
    ),
    compiler_params=pltpu.CompilerParams(
        use_tc_tiling_on_sc=False, needs_layout_passes=False
    ),
)
def _gather(xs_hbm, table_hbm, out_hbm, idx_v, rows_v, tbuf_v, *sems):
    wid = lax.axis_index("s") * NC + lax.axis_index("c")
    c0 = wid * (NGROUP * GC)
    isem = sems[0:2]
    gsem = sems[2:4]
    wsem = sems[4:6]
    iota16 = lax.iota(jnp.int32, 16)

    def chunk_hb(g, b):
        c = c0 + g * GC + b
        return c // CPH, (c % CPH) * CHUNK

    def fire_idx(s, g):
        for b in range(GC):
            h, b0 = chunk_hb(g, b)
            pltpu.async_copy(
                xs_hbm.at[h, pl.ds(b0, CHUNK)], idx_v.at[s, b], isem[s]
            )

    def fire_gathers(s, g):
        for b in range(GC):
            pltpu.make_async_copy(
                xs_hbm.at[0, pl.ds(0, CHUNK)], idx_v.at[s, b], isem[s]
            ).wait()
        for b in range(GC):
            pltpu.async_copy(
                table_hbm.at[idx_v.at[s, b]], rows_v.at[s, b], gsem[s]
            )

    def process(s, g, drain_wb):
        for b in range(GC):
            pltpu.make_async_copy(
                table_hbm.at[pl.ds(0, CHUNK)], rows_v.at[s, b], gsem[s]
            ).wait()
        if drain_wb:
            for b in range(GC):
                for td in range(DIM // 8):
                    pltpu.make_async_copy(
                        tbuf_v.at[s, b, pl.ds(td * 8, 8)],
                        out_hbm.at[0, td, 0],
                        wsem[s],
                    ).wait()
        for b in range(GC):
            rv = rows_v.at[s, b]
            tb = tbuf_v.at[s, b]

            def tbody(d, _, rv=rv, tb=tb):
                for jj in range(8):
                    rowi = iota16 + (jj * 16)
                    coli = jnp.full((16,), 0, jnp.int32) + d
                    v = plsc.load_gather(rv, [rowi, coli])
                    tb[d, pl.ds(jj * 16, 16)] = v
                return _

            lax.fori_loop(0, DIM, tbody, 0)
            h, b0 = chunk_hb(g, b)
            tb_col = b0 // CHUNK
            for td in range(DIM // 8):
                pltpu.async_copy(
                    tbuf_v.at[s, b, pl.ds(td * 8, 8)],
                    out_hbm.at[h, td, tb_col],
                    wsem[s],
                )

    def drain_wb_final(s):
        for b in range(GC):
            for td in range(DIM // 8):
                pltpu.make_async_copy(
                    tbuf_v.at[s, b, pl.ds(td * 8, 8)],
                    out_hbm.at[0, td, 0],
                    wsem[s],
                ).wait()

    fire_idx(0, 0)
    fire_gathers(0, 0)
    fire_idx(1, 1)
    fire_gathers(1, 1)
    process(0, 0, drain_wb=False)
    fire_idx(0, 2)
    fire_gathers(0, 2)
    process(1, 1, drain_wb=False)
    fire_idx(1, 3)

    def steady(k, _):
        g = 3 + 2 * k
        fire_gathers(1, g)
        process(0, g - 1, drain_wb=True)
        fire_idx(0, g + 1)
        fire_gathers(0, g + 1)
        process(1, g, drain_wb=True)
        fire_idx(1, g + 2)
        return _

    lax.fori_loop(0, (NGROUP - 4) // 2, steady, 0)

    g = NGROUP - 1
    fire_gathers(1, g)
    process(0, g - 1, drain_wb=True)
    process(1, g, drain_wb=True)
    drain_wb_final(0)
    drain_wb_final(1)


def kernel(xs, table):
    out_t = _gather(xs.T.astype(jnp.int32), table)
    out = jnp.transpose(out_t, (2, 4, 0, 1, 3))
    return out.reshape(BATCH, HIST, DIM)
